# baseline (speedup 1.0000x reference)
import numpy as np
import jax
import jax.numpy as jnp
from jax import lax
from jax.experimental import pallas as pl
from jax.experimental.pallas import tpu as pltpu
from jax.experimental.pallas import tpu_sc as plsc

NBOX = 32
NANCH = 20000
PAD = 20480
ROWS = PAD // 128
NTILE = 16
CHUNK = PAD // NTILE
VITER = CHUNK // 16


def _rotl(x, r):
    r = np.uint32(r)
    return ((x << r) | (x >> np.uint32(32 - r))).astype(np.uint32)


def _tf2x32(k1, k2, x1, x2):
    x = [np.asarray(x1, np.uint32).copy(), np.asarray(x2, np.uint32).copy()]
    ks = [np.uint32(k1), np.uint32(k2),
          np.uint32(np.uint32(k1) ^ np.uint32(k2) ^ np.uint32(0x1BD11BDA))]
    rotations = [[13, 15, 26, 6], [17, 29, 16, 24]]
    x[0] = (x[0] + ks[0]).astype(np.uint32)
    x[1] = (x[1] + ks[1]).astype(np.uint32)
    for i in range(5):
        for r in rotations[i % 2]:
            x[0] = (x[0] + x[1]).astype(np.uint32)
            x[1] = _rotl(x[1], r)
            x[1] = x[1] ^ x[0]
        x[0] = (x[0] + ks[(i + 1) % 3]).astype(np.uint32)
        x[1] = (x[1] + ks[(i + 2) % 3] + np.uint32(i + 1)).astype(np.uint32)
    return x[0], x[1]


def _np_fold_in(key, data):
    b1, b2 = _tf2x32(key[0], key[1], np.zeros(1, np.uint32),
                     np.array([data], np.uint32))
    return (b1[0], b2[0])


def _np_split2(key):
    b1, b2 = _tf2x32(key[0], key[1], np.zeros(2, np.uint32),
                     np.arange(2, dtype=np.uint32))
    return (b1[0], b2[0]), (b1[1], b2[1])


def _np_bits(key, n):
    b1, b2 = _tf2x32(key[0], key[1], np.zeros(n, np.uint32),
                     np.arange(n, dtype=np.uint32))
    return b1 ^ b2


def _build_tables():
    out = []
    for cls in (0, 1):
        k0 = _np_fold_in((np.uint32(0), np.uint32(42)), cls)
        k1, sub1 = _np_split2(k0)
        bits1 = _np_bits(sub1, NANCH)
        _, sub2 = _np_split2(k1)
        bits2 = _np_bits(sub2, NANCH)
        for b in (bits1, bits2):
            sigma = np.argsort(b, kind="stable").astype(np.int32)
            g = np.empty(NANCH, np.int32)
            g[sigma] = np.arange(NANCH, dtype=np.int32)
            sig_pad = np.full(PAD, 2**30, np.int32)
            sig_pad[:NANCH] = sigma
            g_pad = np.zeros(PAD, np.int32)
            g_pad[:NANCH] = g
            out.append(sig_pad)
            out.append(g_pad)
    return np.stack(out)


_TABLES = _build_tables()
S1P, G1P, S2P, G2P, S1N, G1N, S2N, G2N = range(8)


def _iou_j(bbox_ref, j, ax1, ay1, ax2, ay2, area1):
    bx1 = bbox_ref[j, 0]
    by1 = bbox_ref[j, 1]
    bx2 = bbox_ref[j, 2]
    by2 = bbox_ref[j, 3]
    ix = jnp.maximum(jnp.minimum(ax2, bx2) - jnp.maximum(ax1, bx1), 0.0)
    iy = jnp.maximum(jnp.minimum(ay2, by2) - jnp.maximum(ay1, by1), 0.0)
    ia = ix * iy
    union = area1 + (bx2 - bx1) * (by2 - by1) - ia
    return ia / union, (bx1, by1, bx2, by2)


def _tc_a_body(bbox_ref, a_ref, label_ref, win_ref):
    ax1 = a_ref[0]
    ay1 = a_ref[1]
    ax2 = a_ref[2]
    ay2 = a_ref[3]
    keep = (ax1 >= 0.0) & (ay1 >= 0.0) & (ax2 < 1.0) & (ay2 < 1.0)
    area1 = (ax2 - ax1) * (ay2 - ay1)
    ridx = (lax.broadcasted_iota(jnp.int32, (ROWS, 128), 0) * 128
            + lax.broadcasted_iota(jnp.int32, (ROWS, 128), 1))
    best = jnp.full((ROWS, 128), -jnp.inf, jnp.float32)
    M = jnp.full((NBOX, 128), -2.0, jnp.float32)
    I = jnp.full((NBOX, 128), PAD, jnp.int32)
    mrow = lax.broadcasted_iota(jnp.int32, (NBOX, 128), 0)
    for j in range(NBOX):
        iou, _ = _iou_j(bbox_ref, j, ax1, ay1, ax2, ay2, area1)
        best = jnp.maximum(best, iou)
        miou = jnp.where(keep, iou, -1.0)
        lanemax = jnp.max(miou, axis=0, keepdims=True)
        cand = jnp.where(miou == lanemax, ridx, PAD)
        laneidx = jnp.min(cand, axis=0, keepdims=True)
        M = jnp.where(mrow == j, lanemax, M)
        I = jnp.where(mrow == j, laneidx, I)
    bmax = jnp.max(M, axis=1, keepdims=True)
    wvec = jnp.min(jnp.where(M == bmax, I, PAD), axis=1, keepdims=True)
    label = jnp.where(keep & (best < 0.3), 0, -1)
    label = jnp.where(keep & (best > 0.7), 1, label)
    label_ref[...] = label
    win_ref[...] = jnp.broadcast_to(wvec, (NBOX, 128))


def _tc_b_body(bbox_ref, a_ref, loc_ref):
    ax1 = a_ref[0]
    ay1 = a_ref[1]
    ax2 = a_ref[2]
    ay2 = a_ref[3]
    keep = (ax1 >= 0.0) & (ay1 >= 0.0) & (ax2 < 1.0) & (ay2 < 1.0)
    area1 = (ax2 - ax1) * (ay2 - ay1)
    best = jnp.full((ROWS, 128), -jnp.inf, jnp.float32)
    zero = jnp.zeros((ROWS, 128), jnp.float32)
    sx1, sy1, sx2, sy2 = zero, zero, zero, zero
    for j in range(NBOX):
        iou, (bx1, by1, bx2, by2) = _iou_j(bbox_ref, j, ax1, ay1, ax2, ay2,
                                           area1)
        better = iou > best
        best = jnp.where(better, iou, best)
        sx1 = jnp.where(better, bx1, sx1)
        sy1 = jnp.where(better, by1, sy1)
        sx2 = jnp.where(better, bx2, sx2)
        sy2 = jnp.where(better, by2, sy2)
    aw = ax2 - ax1
    ah = ay2 - ay1
    fkeep = keep.astype(jnp.float32)
    loc_ref[0] = jnp.where(keep, ((sx1 + sx2) / 2.0 - (ax1 + ax2) / 2.0) / aw, 0.0)
    loc_ref[1] = jnp.where(keep, ((sy1 + sy2) / 2.0 - (ay1 + ay2) / 2.0) / ah, 0.0)
    loc_ref[2] = fkeep * jnp.log(jnp.where(keep, (sx2 - sx1) / aw, 1.0))
    loc_ref[3] = fkeep * jnp.log(jnp.where(keep, (sy2 - sy1) / ah, 1.0))


_tc_a_call = pl.pallas_call(
    _tc_a_body,
    in_specs=[
        pl.BlockSpec(memory_space=pltpu.SMEM),
        pl.BlockSpec(memory_space=pltpu.VMEM),
    ],
    out_specs=[
        pl.BlockSpec(memory_space=pltpu.VMEM),
        pl.BlockSpec(memory_space=pltpu.VMEM),
    ],
    out_shape=[
        jax.ShapeDtypeStruct((ROWS, 128), jnp.int32),
        jax.ShapeDtypeStruct((NBOX, 128), jnp.int32),
    ],
)

_tc_b_call = pl.pallas_call(
    _tc_b_body,
    in_specs=[
        pl.BlockSpec(memory_space=pltpu.SMEM),
        pl.BlockSpec(memory_space=pltpu.VMEM),
    ],
    out_specs=[pl.BlockSpec(memory_space=pltpu.VMEM)],
    out_shape=[jax.ShapeDtypeStruct((4, ROWS, 128), jnp.float32)],
)


def _splat(x):
    return jnp.zeros((16,), jnp.int32) + x


def _sc_body(lbl_hbm, win_hbm, tbl_hbm, out_hbm, c1s_hbm, c2s_hbm,
             c1ps_hbm, c2ps_hbm,
             lbl_v, win_v, prank_v, nrank_v, siga_v, sigb_v,
             c1p_v, c2p_v, c1n_v, c2n_v,
             bg1_v, bc1_v, bg2_v, bc2_v,
             row_v, tbl_v,
             spm_cnt, spm_tot,
             sem1, sem2):
    t = lax.axis_index("s")
    base = t * CHUNK
    iota = lax.iota(jnp.int32, 16)
    zeros = _splat(0)
    ones = _splat(1)

    cp1 = pltpu.async_copy(tbl_hbm.at[G1N], bg1_v, sem1)
    cp2 = pltpu.async_copy(tbl_hbm.at[G2N], bg2_v, sem2)

    pltpu.sync_copy(lbl_hbm.at[pl.ds(base, CHUNK)], lbl_v)
    pltpu.sync_copy(win_hbm, win_v)

    for h in range(2):
        w = plsc.load_gather(win_v, [iota + _splat(h * 16), zeros])
        loc = w - _splat(base)
        msk = (loc >= zeros) & (loc < _splat(CHUNK))
        locc = jnp.minimum(jnp.maximum(loc, zeros), _splat(CHUNK - 1))
        plsc.store_scatter(lbl_v, [locc], ones, mask=msk)

    def cnt_body(i, carry):
        cp, cn = carry
        la = lbl_v[pl.ds(i * 32, 16)]
        lb = lbl_v[pl.ds(i * 32 + 16, 16)]
        mpa = la == ones
        mna = la == zeros
        mpb = lb == ones
        mnb = lb == zeros
        pa = plsc.all_reduce_population_count(mpa)
        na = plsc.all_reduce_population_count(mna)
        pb = plsc.all_reduce_population_count(mpb)
        nb = plsc.all_reduce_population_count(mnb)
        prank_v[pl.ds(i * 32, 16)] = plsc.cumsum(jnp.where(mpa, ones, zeros)) + cp
        nrank_v[pl.ds(i * 32, 16)] = plsc.cumsum(jnp.where(mna, ones, zeros)) + cn
        prank_v[pl.ds(i * 32 + 16, 16)] = (
            plsc.cumsum(jnp.where(mpb, ones, zeros)) + cp + pa)
        nrank_v[pl.ds(i * 32 + 16, 16)] = (
            plsc.cumsum(jnp.where(mnb, ones, zeros)) + cn + na)
        return (cp + pa + pb, cn + na + nb)

    cp, cn = lax.fori_loop(0, VITER // 2, cnt_body, (zeros, zeros))
    my_pos = jnp.max(cp)
    my_neg = jnp.max(cn)
    row_v[...] = (jnp.where(iota == zeros, _splat(my_pos), zeros)
                  + jnp.where(iota == ones, _splat(my_neg), zeros))
    pltpu.sync_copy(row_v, spm_cnt.at[t])
    plsc.subcore_barrier()

    pltpu.sync_copy(spm_cnt, tbl_v)
    pc = plsc.load_gather(tbl_v, [iota, zeros])
    nc = plsc.load_gather(tbl_v, [iota, ones])
    n_pos = jnp.sum(pc)
    n_neg = jnp.sum(nc)
    tv = _splat(t)
    off_pos = jnp.sum(jnp.where(iota < tv, pc, zeros))
    off_neg = jnp.sum(jnp.where(iota < tv, nc, zeros))
    pos_active = n_pos > 128

    def c_pair(sa, sb, ca_v, cb_v, n):
        pltpu.sync_copy(tbl_hbm.at[sa, pl.ds(base, CHUNK)], siga_v)
        pltpu.sync_copy(tbl_hbm.at[sb, pl.ds(base, CHUNK)], sigb_v)
        nv = _splat(n)

        def body(i, carry):
            ca, cb = carry
            sa0 = siga_v[pl.ds(i * 32, 16)]
            sa1 = siga_v[pl.ds(i * 32 + 16, 16)]
            sb0 = sigb_v[pl.ds(i * 32, 16)]
            sb1 = sigb_v[pl.ds(i * 32 + 16, 16)]
            ma0 = sa0 < nv
            ma1 = sa1 < nv
            mb0 = sb0 < nv
            mb1 = sb1 < nv
            pa0 = plsc.all_reduce_population_count(ma0)
            pa1 = plsc.all_reduce_population_count(ma1)
            pb0 = plsc.all_reduce_population_count(mb0)
            pb1 = plsc.all_reduce_population_count(mb1)
            ia0 = jnp.where(ma0, ones, zeros)
            ia1 = jnp.where(ma1, ones, zeros)
            ib0 = jnp.where(mb0, ones, zeros)
            ib1 = jnp.where(mb1, ones, zeros)
            ca_v[pl.ds(i * 32, 16)] = plsc.cumsum(ia0) + ca - ia0
            cb_v[pl.ds(i * 32, 16)] = plsc.cumsum(ib0) + cb - ib0
            ca_v[pl.ds(i * 32 + 16, 16)] = plsc.cumsum(ia1) + (ca + pa0) - ia1
            cb_v[pl.ds(i * 32 + 16, 16)] = plsc.cumsum(ib1) + (cb + pb0) - ib1
            return (ca + pa0 + pa1, cb + pb0 + pb1)

        ta, tb = lax.fori_loop(0, VITER // 2, body, (zeros, zeros))
        return jnp.max(ta), jnp.max(tb)

    t1n, t2n = c_pair(S1N, S2N, c1n_v, c2n_v, n_neg)
    row_v[...] = (jnp.where(iota == _splat(2), _splat(t1n), zeros)
                  + jnp.where(iota == _splat(3), _splat(t2n), zeros))

    @pl.when(pos_active)
    def _():
        t1p, t2p = c_pair(S1P, S2P, c1p_v, c2p_v, n_pos)
        row_v[...] = (row_v[...]
                      + jnp.where(iota == zeros, _splat(t1p), zeros)
                      + jnp.where(iota == ones, _splat(t2p), zeros))

    pltpu.sync_copy(row_v, spm_tot.at[t])
    plsc.subcore_barrier()

    pltpu.sync_copy(spm_tot, tbl_v)

    def off_of(col):
        tc = plsc.load_gather(tbl_v, [iota, _splat(col)])
        return _splat(jnp.sum(jnp.where(iota < tv, tc, zeros)))

    o1n = off_of(2)
    o2n = off_of(3)

    def pub_body(i, carry):
        c1n_v[pl.ds(i * 16, 16)] = c1n_v[pl.ds(i * 16, 16)] + o1n
        c2n_v[pl.ds(i * 16, 16)] = c2n_v[pl.ds(i * 16, 16)] + o2n
        return carry

    lax.fori_loop(0, VITER, pub_body, 0)
    pltpu.sync_copy(c1n_v, c1s_hbm.at[pl.ds(base, CHUNK)])
    pltpu.sync_copy(c2n_v, c2s_hbm.at[pl.ds(base, CHUNK)])

    @pl.when(pos_active)
    def _():
        o1p = off_of(0)
        o2p = off_of(1)

        def body(i, carry):
            c1p_v[pl.ds(i * 16, 16)] = c1p_v[pl.ds(i * 16, 16)] + o1p
            c2p_v[pl.ds(i * 16, 16)] = c2p_v[pl.ds(i * 16, 16)] + o2p
            return carry

        lax.fori_loop(0, VITER, body, 0)
        pltpu.sync_copy(c1p_v, c1ps_hbm.at[pl.ds(base, CHUNK)])
        pltpu.sync_copy(c2p_v, c2ps_hbm.at[pl.ds(base, CHUNK)])

    plsc.subcore_barrier()

    maxi = _splat(PAD - 1)
    cp1.wait()
    cp2.wait()

    def drop_loop(rank_v, n, off, lblval, start):
        nvec = _splat(n)
        startv = _splat(start)
        offv = _splat(off)
        lv = _splat(lblval)

        two_round = nvec > _splat(1625)
        neg1 = _splat(-1)

        def chain(r):
            rc = jnp.minimum(jnp.maximum(r, zeros), maxi)
            a = plsc.load_gather(bg1_v, [rc])
            p1 = plsc.load_gather(bc1_v, [a])
            b = plsc.load_gather(bg2_v, [jnp.minimum(p1, maxi)])
            v2 = plsc.load_gather(bc2_v, [b])
            return jnp.where(two_round, v2, p1)

        def body(i, carry):
            l0 = lbl_v[pl.ds(i * 32, 16)]
            l1 = lbl_v[pl.ds(i * 32 + 16, 16)]
            r0 = rank_v[pl.ds(i * 32, 16)] - ones + offv
            r1 = rank_v[pl.ds(i * 32 + 16, 16)] - ones + offv
            v0 = chain(r0)
            v1 = chain(r1)
            d0 = (l0 == lv) & (v0 >= startv)
            d1 = (l1 == lv) & (v1 >= startv)
            lbl_v[pl.ds(i * 32, 16)] = jnp.where(d0, neg1, l0)
            lbl_v[pl.ds(i * 32 + 16, 16)] = jnp.where(d1, neg1, l1)
            return carry

        lax.fori_loop(0, VITER // 2, body, 0)

    @pl.when(pos_active)
    def _():
        pltpu.sync_copy(tbl_hbm.at[G1P], bg1_v)
        pltpu.sync_copy(c1ps_hbm, bc1_v)
        pltpu.sync_copy(tbl_hbm.at[G2P], bg2_v)
        pltpu.sync_copy(c2ps_hbm, bc2_v)
        drop_loop(prank_v, n_pos, off_pos, 1, 128)
        pltpu.sync_copy(tbl_hbm.at[G1N], bg1_v)
        pltpu.sync_copy(tbl_hbm.at[G2N], bg2_v)

    pltpu.sync_copy(c1s_hbm, bc1_v)
    pltpu.sync_copy(c2s_hbm, bc2_v)
    s = 256 - n_pos - n_neg
    start_lt = jnp.where(s >= 0, jnp.minimum(s, n_neg),
                         jnp.maximum(n_neg + s, 0))
    start_neg = jnp.where(n_pos >= 128, 128, start_lt)

    @pl.when(n_neg > 128)
    def _():
        drop_loop(nrank_v, n_neg, off_neg, 0, start_neg)

    pltpu.sync_copy(lbl_v, out_hbm.at[pl.ds(base, CHUNK)])


_sc_call = pl.kernel(
    _sc_body,
    mesh=plsc.VectorSubcoreMesh(core_axis_name="c", subcore_axis_name="s",
                                num_cores=1),
    out_type=[jax.ShapeDtypeStruct((PAD,), jnp.int32)] * 5,
    compiler_params=pltpu.CompilerParams(needs_layout_passes=False),
    scratch_types=[
        pltpu.VMEM((CHUNK,), jnp.int32),
        pltpu.VMEM((NBOX, 128), jnp.int32),
        pltpu.VMEM((CHUNK,), jnp.int32),
        pltpu.VMEM((CHUNK,), jnp.int32),
        pltpu.VMEM((CHUNK,), jnp.int32),
        pltpu.VMEM((CHUNK,), jnp.int32),
        pltpu.VMEM((CHUNK,), jnp.int32),
        pltpu.VMEM((CHUNK,), jnp.int32),
        pltpu.VMEM((CHUNK,), jnp.int32),
        pltpu.VMEM((CHUNK,), jnp.int32),
        pltpu.VMEM((PAD,), jnp.int32),
        pltpu.VMEM((PAD,), jnp.int32),
        pltpu.VMEM((PAD,), jnp.int32),
        pltpu.VMEM((PAD,), jnp.int32),
        pltpu.VMEM((16,), jnp.int32),
        pltpu.VMEM((16, 16), jnp.int32),
        pltpu.VMEM_SHARED((NTILE, 16), jnp.int32),
        pltpu.VMEM_SHARED((NTILE, 16), jnp.int32),
        pltpu.SemaphoreType.DMA,
        pltpu.SemaphoreType.DMA,
    ],
)


def kernel(bbox, anchor):
    bbox = bbox.astype(jnp.float32)
    anchor = anchor.astype(jnp.float32)
    pads = jnp.tile(jnp.array([[2.0], [2.0], [2.1], [2.1]], jnp.float32),
                    (1, PAD - NANCH))
    acoord = jnp.concatenate([anchor.T, pads], axis=1).reshape(4, ROWS, 128)
    label0, winners = _tc_a_call(bbox, acoord)
    cls_pad, _, _, _, _ = _sc_call(label0.reshape(PAD), winners,
                                   jnp.asarray(_TABLES))
    (locp,) = _tc_b_call(bbox, acoord)
    rpn_tg_cls = cls_pad[:NANCH]
    rpn_tg_loc = locp.reshape(4, PAD).T[:NANCH]
    return (rpn_tg_cls, rpn_tg_loc)


# speedup vs baseline: 1.0675x; 1.0675x over previous
import numpy as np
import jax
import jax.numpy as jnp
from jax import lax
from jax.experimental import pallas as pl
from jax.experimental.pallas import tpu as pltpu
from jax.experimental.pallas import tpu_sc as plsc

NBOX = 32
NANCH = 20000
PAD = 20480
ROWS = PAD // 128
NTILE = 16
CHUNK = PAD // NTILE
VITER = CHUNK // 16


def _rotl(x, r):
    r = np.uint32(r)
    return ((x << r) | (x >> np.uint32(32 - r))).astype(np.uint32)


def _tf2x32(k1, k2, x1, x2):
    x = [np.asarray(x1, np.uint32).copy(), np.asarray(x2, np.uint32).copy()]
    ks = [np.uint32(k1), np.uint32(k2),
          np.uint32(np.uint32(k1) ^ np.uint32(k2) ^ np.uint32(0x1BD11BDA))]
    rotations = [[13, 15, 26, 6], [17, 29, 16, 24]]
    x[0] = (x[0] + ks[0]).astype(np.uint32)
    x[1] = (x[1] + ks[1]).astype(np.uint32)
    for i in range(5):
        for r in rotations[i % 2]:
            x[0] = (x[0] + x[1]).astype(np.uint32)
            x[1] = _rotl(x[1], r)
            x[1] = x[1] ^ x[0]
        x[0] = (x[0] + ks[(i + 1) % 3]).astype(np.uint32)
        x[1] = (x[1] + ks[(i + 2) % 3] + np.uint32(i + 1)).astype(np.uint32)
    return x[0], x[1]


def _np_fold_in(key, data):
    b1, b2 = _tf2x32(key[0], key[1], np.zeros(1, np.uint32),
                     np.array([data], np.uint32))
    return (b1[0], b2[0])


def _np_split2(key):
    b1, b2 = _tf2x32(key[0], key[1], np.zeros(2, np.uint32),
                     np.arange(2, dtype=np.uint32))
    return (b1[0], b2[0]), (b1[1], b2[1])


def _np_bits(key, n):
    b1, b2 = _tf2x32(key[0], key[1], np.zeros(n, np.uint32),
                     np.arange(n, dtype=np.uint32))
    return b1 ^ b2


def _build_tables():
    out = []
    for cls in (0, 1):
        k0 = _np_fold_in((np.uint32(0), np.uint32(42)), cls)
        k1, sub1 = _np_split2(k0)
        bits1 = _np_bits(sub1, NANCH)
        _, sub2 = _np_split2(k1)
        bits2 = _np_bits(sub2, NANCH)
        for b in (bits1, bits2):
            sigma = np.argsort(b, kind="stable").astype(np.int32)
            g = np.empty(NANCH, np.int32)
            g[sigma] = np.arange(NANCH, dtype=np.int32)
            sig_pad = np.full(PAD, 2**30, np.int32)
            sig_pad[:NANCH] = sigma
            g_pad = np.zeros(PAD, np.int32)
            g_pad[:NANCH] = g
            out.append(sig_pad)
            out.append(g_pad)
    return np.stack(out)


_TABLES = _build_tables()
S1P, G1P, S2P, G2P, S1N, G1N, S2N, G2N = range(8)


def _iou_j(bbox_ref, j, ax1, ay1, ax2, ay2, area1):
    bx1 = bbox_ref[j, 0]
    by1 = bbox_ref[j, 1]
    bx2 = bbox_ref[j, 2]
    by2 = bbox_ref[j, 3]
    ix = jnp.maximum(jnp.minimum(ax2, bx2) - jnp.maximum(ax1, bx1), 0.0)
    iy = jnp.maximum(jnp.minimum(ay2, by2) - jnp.maximum(ay1, by1), 0.0)
    ia = ix * iy
    union = area1 + (bx2 - bx1) * (by2 - by1) - ia
    return ia / union, (bx1, by1, bx2, by2)


def _tc_a_body(bbox_ref, a_ref, label_ref, win_ref):
    ax1 = a_ref[0]
    ay1 = a_ref[1]
    ax2 = a_ref[2]
    ay2 = a_ref[3]
    keep = (ax1 >= 0.0) & (ay1 >= 0.0) & (ax2 < 1.0) & (ay2 < 1.0)
    area1 = (ax2 - ax1) * (ay2 - ay1)
    ridx = (lax.broadcasted_iota(jnp.int32, (ROWS, 128), 0) * 128
            + lax.broadcasted_iota(jnp.int32, (ROWS, 128), 1))
    best = jnp.full((ROWS, 128), -jnp.inf, jnp.float32)
    M = jnp.full((NBOX, 128), -2.0, jnp.float32)
    I = jnp.full((NBOX, 128), PAD, jnp.int32)
    mrow = lax.broadcasted_iota(jnp.int32, (NBOX, 128), 0)
    for j in range(NBOX):
        iou, _ = _iou_j(bbox_ref, j, ax1, ay1, ax2, ay2, area1)
        best = jnp.maximum(best, iou)
        miou = jnp.where(keep, iou, -1.0)
        lanemax = jnp.max(miou, axis=0, keepdims=True)
        cand = jnp.where(miou == lanemax, ridx, PAD)
        laneidx = jnp.min(cand, axis=0, keepdims=True)
        M = jnp.where(mrow == j, lanemax, M)
        I = jnp.where(mrow == j, laneidx, I)
    bmax = jnp.max(M, axis=1, keepdims=True)
    wvec = jnp.min(jnp.where(M == bmax, I, PAD), axis=1, keepdims=True)
    label = jnp.where(keep & (best < 0.3), 0, -1)
    label = jnp.where(keep & (best > 0.7), 1, label)
    label_ref[...] = label
    win_ref[...] = jnp.broadcast_to(wvec, (NBOX, 128))


def _tc_b_body(bbox_ref, a_ref, loc_ref):
    ax1 = a_ref[0]
    ay1 = a_ref[1]
    ax2 = a_ref[2]
    ay2 = a_ref[3]
    keep = (ax1 >= 0.0) & (ay1 >= 0.0) & (ax2 < 1.0) & (ay2 < 1.0)
    area1 = (ax2 - ax1) * (ay2 - ay1)
    best = jnp.full((ROWS, 128), -jnp.inf, jnp.float32)
    zero = jnp.zeros((ROWS, 128), jnp.float32)
    sx1, sy1, sx2, sy2 = zero, zero, zero, zero
    for j in range(NBOX):
        iou, (bx1, by1, bx2, by2) = _iou_j(bbox_ref, j, ax1, ay1, ax2, ay2,
                                           area1)
        better = iou > best
        best = jnp.where(better, iou, best)
        sx1 = jnp.where(better, bx1, sx1)
        sy1 = jnp.where(better, by1, sy1)
        sx2 = jnp.where(better, bx2, sx2)
        sy2 = jnp.where(better, by2, sy2)
    aw = ax2 - ax1
    ah = ay2 - ay1
    fkeep = keep.astype(jnp.float32)
    loc_ref[0] = jnp.where(keep, ((sx1 + sx2) / 2.0 - (ax1 + ax2) / 2.0) / aw, 0.0)
    loc_ref[1] = jnp.where(keep, ((sy1 + sy2) / 2.0 - (ay1 + ay2) / 2.0) / ah, 0.0)
    loc_ref[2] = fkeep * jnp.log(jnp.where(keep, (sx2 - sx1) / aw, 1.0))
    loc_ref[3] = fkeep * jnp.log(jnp.where(keep, (sy2 - sy1) / ah, 1.0))


_tc_a_call = pl.pallas_call(
    _tc_a_body,
    in_specs=[
        pl.BlockSpec(memory_space=pltpu.SMEM),
        pl.BlockSpec(memory_space=pltpu.VMEM),
    ],
    out_specs=[
        pl.BlockSpec(memory_space=pltpu.VMEM),
        pl.BlockSpec(memory_space=pltpu.VMEM),
    ],
    out_shape=[
        jax.ShapeDtypeStruct((ROWS, 128), jnp.int32),
        jax.ShapeDtypeStruct((NBOX, 128), jnp.int32),
    ],
)

_tc_b_call = pl.pallas_call(
    _tc_b_body,
    in_specs=[
        pl.BlockSpec(memory_space=pltpu.SMEM),
        pl.BlockSpec(memory_space=pltpu.VMEM),
    ],
    out_specs=[pl.BlockSpec(memory_space=pltpu.VMEM)],
    out_shape=[jax.ShapeDtypeStruct((4, ROWS, 128), jnp.float32)],
)


def _splat(x):
    return jnp.zeros((16,), jnp.int32) + x


def _sc_body(lbl_hbm, win_hbm, tbl_hbm, out_hbm,
             lbl_v, win_v, prank_v, nrank_v, siga_v, sigb_v,
             c1p_v, c2p_v, c1n_v, c2n_v,
             bg1_v, bc1_v, bg2_v, bc2_v,
             row_v, tbl_v,
             spm_cnt, spm_tot, spm_c1, spm_c2, spm_c1p, spm_c2p,
             sem1, sem2):
    t = lax.axis_index("s")
    base = t * CHUNK
    iota = lax.iota(jnp.int32, 16)
    zeros = _splat(0)
    ones = _splat(1)

    cp1 = pltpu.async_copy(tbl_hbm.at[G1N], bg1_v, sem1)
    cp2 = pltpu.async_copy(tbl_hbm.at[G2N], bg2_v, sem2)

    pltpu.sync_copy(lbl_hbm.at[pl.ds(base, CHUNK)], lbl_v)
    pltpu.sync_copy(win_hbm, win_v)

    for h in range(2):
        w = plsc.load_gather(win_v, [iota + _splat(h * 16), zeros])
        loc = w - _splat(base)
        msk = (loc >= zeros) & (loc < _splat(CHUNK))
        locc = jnp.minimum(jnp.maximum(loc, zeros), _splat(CHUNK - 1))
        plsc.store_scatter(lbl_v, [locc], ones, mask=msk)

    def cnt_body(i, carry):
        cp, cn = carry
        la = lbl_v[pl.ds(i * 32, 16)]
        lb = lbl_v[pl.ds(i * 32 + 16, 16)]
        mpa = la == ones
        mna = la == zeros
        mpb = lb == ones
        mnb = lb == zeros
        pa = plsc.all_reduce_population_count(mpa)
        na = plsc.all_reduce_population_count(mna)
        pb = plsc.all_reduce_population_count(mpb)
        nb = plsc.all_reduce_population_count(mnb)
        prank_v[pl.ds(i * 32, 16)] = plsc.cumsum(jnp.where(mpa, ones, zeros)) + cp
        nrank_v[pl.ds(i * 32, 16)] = plsc.cumsum(jnp.where(mna, ones, zeros)) + cn
        prank_v[pl.ds(i * 32 + 16, 16)] = (
            plsc.cumsum(jnp.where(mpb, ones, zeros)) + cp + pa)
        nrank_v[pl.ds(i * 32 + 16, 16)] = (
            plsc.cumsum(jnp.where(mnb, ones, zeros)) + cn + na)
        return (cp + pa + pb, cn + na + nb)

    cp, cn = lax.fori_loop(0, VITER // 2, cnt_body, (zeros, zeros))
    my_pos = jnp.max(cp)
    my_neg = jnp.max(cn)
    row_v[...] = (jnp.where(iota == zeros, _splat(my_pos), zeros)
                  + jnp.where(iota == ones, _splat(my_neg), zeros))
    pltpu.sync_copy(row_v, spm_cnt.at[t])
    plsc.subcore_barrier()

    pltpu.sync_copy(spm_cnt, tbl_v)
    pc = plsc.load_gather(tbl_v, [iota, zeros])
    nc = plsc.load_gather(tbl_v, [iota, ones])
    n_pos = jnp.sum(pc)
    n_neg = jnp.sum(nc)
    tv = _splat(t)
    off_pos = jnp.sum(jnp.where(iota < tv, pc, zeros))
    off_neg = jnp.sum(jnp.where(iota < tv, nc, zeros))
    pos_active = n_pos > 128

    def c_pair(sa, sb, ca_v, cb_v, n):
        pltpu.sync_copy(tbl_hbm.at[sa, pl.ds(base, CHUNK)], siga_v)
        pltpu.sync_copy(tbl_hbm.at[sb, pl.ds(base, CHUNK)], sigb_v)
        nv = _splat(n)

        def body(i, carry):
            ca, cb = carry
            sa0 = siga_v[pl.ds(i * 32, 16)]
            sa1 = siga_v[pl.ds(i * 32 + 16, 16)]
            sb0 = sigb_v[pl.ds(i * 32, 16)]
            sb1 = sigb_v[pl.ds(i * 32 + 16, 16)]
            ma0 = sa0 < nv
            ma1 = sa1 < nv
            mb0 = sb0 < nv
            mb1 = sb1 < nv
            pa0 = plsc.all_reduce_population_count(ma0)
            pa1 = plsc.all_reduce_population_count(ma1)
            pb0 = plsc.all_reduce_population_count(mb0)
            pb1 = plsc.all_reduce_population_count(mb1)
            ia0 = jnp.where(ma0, ones, zeros)
            ia1 = jnp.where(ma1, ones, zeros)
            ib0 = jnp.where(mb0, ones, zeros)
            ib1 = jnp.where(mb1, ones, zeros)
            ca_v[pl.ds(i * 32, 16)] = plsc.cumsum(ia0) + ca - ia0
            cb_v[pl.ds(i * 32, 16)] = plsc.cumsum(ib0) + cb - ib0
            ca_v[pl.ds(i * 32 + 16, 16)] = plsc.cumsum(ia1) + (ca + pa0) - ia1
            cb_v[pl.ds(i * 32 + 16, 16)] = plsc.cumsum(ib1) + (cb + pb0) - ib1
            return (ca + pa0 + pa1, cb + pb0 + pb1)

        ta, tb = lax.fori_loop(0, VITER // 2, body, (zeros, zeros))
        return jnp.max(ta), jnp.max(tb)

    t1n, t2n = c_pair(S1N, S2N, c1n_v, c2n_v, n_neg)
    row_v[...] = (jnp.where(iota == _splat(2), _splat(t1n), zeros)
                  + jnp.where(iota == _splat(3), _splat(t2n), zeros))

    @pl.when(pos_active)
    def _():
        t1p, t2p = c_pair(S1P, S2P, c1p_v, c2p_v, n_pos)
        row_v[...] = (row_v[...]
                      + jnp.where(iota == zeros, _splat(t1p), zeros)
                      + jnp.where(iota == ones, _splat(t2p), zeros))

    pltpu.sync_copy(row_v, spm_tot.at[t])
    plsc.subcore_barrier()

    pltpu.sync_copy(spm_tot, tbl_v)

    def off_of(col):
        tc = plsc.load_gather(tbl_v, [iota, _splat(col)])
        return _splat(jnp.sum(jnp.where(iota < tv, tc, zeros)))

    o1n = off_of(2)
    o2n = off_of(3)

    def pub_body(i, carry):
        c1n_v[pl.ds(i * 16, 16)] = c1n_v[pl.ds(i * 16, 16)] + o1n
        c2n_v[pl.ds(i * 16, 16)] = c2n_v[pl.ds(i * 16, 16)] + o2n
        return carry

    lax.fori_loop(0, VITER, pub_body, 0)
    pltpu.sync_copy(c1n_v, spm_c1.at[pl.ds(base, CHUNK)])
    pltpu.sync_copy(c2n_v, spm_c2.at[pl.ds(base, CHUNK)])

    @pl.when(pos_active)
    def _():
        o1p = off_of(0)
        o2p = off_of(1)

        def body(i, carry):
            c1p_v[pl.ds(i * 16, 16)] = c1p_v[pl.ds(i * 16, 16)] + o1p
            c2p_v[pl.ds(i * 16, 16)] = c2p_v[pl.ds(i * 16, 16)] + o2p
            return carry

        lax.fori_loop(0, VITER, body, 0)
        pltpu.sync_copy(c1p_v, spm_c1p.at[pl.ds(base, CHUNK)])
        pltpu.sync_copy(c2p_v, spm_c2p.at[pl.ds(base, CHUNK)])

    plsc.subcore_barrier()

    maxi = _splat(PAD - 1)
    cp1.wait()
    cp2.wait()

    def drop_loop(rank_v, n, off, lblval, start):
        nvec = _splat(n)
        startv = _splat(start)
        offv = _splat(off)
        lv = _splat(lblval)

        two_round = nvec > _splat(1625)
        neg1 = _splat(-1)

        def chain(r):
            rc = jnp.minimum(jnp.maximum(r, zeros), maxi)
            a = plsc.load_gather(bg1_v, [rc])
            p1 = plsc.load_gather(bc1_v, [a])
            b = plsc.load_gather(bg2_v, [jnp.minimum(p1, maxi)])
            v2 = plsc.load_gather(bc2_v, [b])
            return jnp.where(two_round, v2, p1)

        def body(i, carry):
            l0 = lbl_v[pl.ds(i * 32, 16)]
            l1 = lbl_v[pl.ds(i * 32 + 16, 16)]
            r0 = rank_v[pl.ds(i * 32, 16)] - ones + offv
            r1 = rank_v[pl.ds(i * 32 + 16, 16)] - ones + offv
            v0 = chain(r0)
            v1 = chain(r1)
            d0 = (l0 == lv) & (v0 >= startv)
            d1 = (l1 == lv) & (v1 >= startv)
            lbl_v[pl.ds(i * 32, 16)] = jnp.where(d0, neg1, l0)
            lbl_v[pl.ds(i * 32 + 16, 16)] = jnp.where(d1, neg1, l1)
            return carry

        lax.fori_loop(0, VITER // 2, body, 0)

    @pl.when(pos_active)
    def _():
        pltpu.sync_copy(tbl_hbm.at[G1P], bg1_v)
        pltpu.sync_copy(spm_c1p, bc1_v)
        pltpu.sync_copy(tbl_hbm.at[G2P], bg2_v)
        pltpu.sync_copy(spm_c2p, bc2_v)
        drop_loop(prank_v, n_pos, off_pos, 1, 128)
        pltpu.sync_copy(tbl_hbm.at[G1N], bg1_v)
        pltpu.sync_copy(tbl_hbm.at[G2N], bg2_v)

    pltpu.sync_copy(spm_c1, bc1_v)
    pltpu.sync_copy(spm_c2, bc2_v)
    s = 256 - n_pos - n_neg
    start_lt = jnp.where(s >= 0, jnp.minimum(s, n_neg),
                         jnp.maximum(n_neg + s, 0))
    start_neg = jnp.where(n_pos >= 128, 128, start_lt)

    @pl.when(n_neg > 128)
    def _():
        drop_loop(nrank_v, n_neg, off_neg, 0, start_neg)

    pltpu.sync_copy(lbl_v, out_hbm.at[pl.ds(base, CHUNK)])


_sc_call = pl.kernel(
    _sc_body,
    mesh=plsc.VectorSubcoreMesh(core_axis_name="c", subcore_axis_name="s",
                                num_cores=1),
    out_type=jax.ShapeDtypeStruct((PAD,), jnp.int32),
    compiler_params=pltpu.CompilerParams(needs_layout_passes=False),
    scratch_types=[
        pltpu.VMEM((CHUNK,), jnp.int32),
        pltpu.VMEM((NBOX, 128), jnp.int32),
        pltpu.VMEM((CHUNK,), jnp.int32),
        pltpu.VMEM((CHUNK,), jnp.int32),
        pltpu.VMEM((CHUNK,), jnp.int32),
        pltpu.VMEM((CHUNK,), jnp.int32),
        pltpu.VMEM((CHUNK,), jnp.int32),
        pltpu.VMEM((CHUNK,), jnp.int32),
        pltpu.VMEM((CHUNK,), jnp.int32),
        pltpu.VMEM((CHUNK,), jnp.int32),
        pltpu.VMEM((PAD,), jnp.int32),
        pltpu.VMEM((PAD,), jnp.int32),
        pltpu.VMEM((PAD,), jnp.int32),
        pltpu.VMEM((PAD,), jnp.int32),
        pltpu.VMEM((16,), jnp.int32),
        pltpu.VMEM((16, 16), jnp.int32),
        pltpu.VMEM_SHARED((NTILE, 16), jnp.int32),
        pltpu.VMEM_SHARED((NTILE, 16), jnp.int32),
        pltpu.VMEM_SHARED((PAD,), jnp.int32),
        pltpu.VMEM_SHARED((PAD,), jnp.int32),
        pltpu.VMEM_SHARED((PAD,), jnp.int32),
        pltpu.VMEM_SHARED((PAD,), jnp.int32),
        pltpu.SemaphoreType.DMA,
        pltpu.SemaphoreType.DMA,
    ],
)


def kernel(bbox, anchor):
    bbox = bbox.astype(jnp.float32)
    anchor = anchor.astype(jnp.float32)
    pads = jnp.tile(jnp.array([[2.0], [2.0], [2.1], [2.1]], jnp.float32),
                    (1, PAD - NANCH))
    acoord = jnp.concatenate([anchor.T, pads], axis=1).reshape(4, ROWS, 128)
    label0, winners = _tc_a_call(bbox, acoord)
    cls_pad = _sc_call(label0.reshape(PAD), winners, jnp.asarray(_TABLES))
    (locp,) = _tc_b_call(bbox, acoord)
    rpn_tg_cls = cls_pad[:NANCH]
    rpn_tg_loc = locp.reshape(4, PAD).T[:NANCH]
    return (rpn_tg_cls, rpn_tg_loc)


# speedup vs baseline: 1.1420x; 1.0697x over previous
import numpy as np
import jax
import jax.numpy as jnp
from jax import lax
from jax.experimental import pallas as pl
from jax.experimental.pallas import tpu as pltpu
from jax.experimental.pallas import tpu_sc as plsc

NBOX = 32
NANCH = 20000
PAD = 20480
ROWS = PAD // 128
NTILE = 16
CHUNK = PAD // NTILE
VITER = CHUNK // 16


def _rotl(x, r):
    r = np.uint32(r)
    return ((x << r) | (x >> np.uint32(32 - r))).astype(np.uint32)


def _tf2x32(k1, k2, x1, x2):
    x = [np.asarray(x1, np.uint32).copy(), np.asarray(x2, np.uint32).copy()]
    ks = [np.uint32(k1), np.uint32(k2),
          np.uint32(np.uint32(k1) ^ np.uint32(k2) ^ np.uint32(0x1BD11BDA))]
    rotations = [[13, 15, 26, 6], [17, 29, 16, 24]]
    x[0] = (x[0] + ks[0]).astype(np.uint32)
    x[1] = (x[1] + ks[1]).astype(np.uint32)
    for i in range(5):
        for r in rotations[i % 2]:
            x[0] = (x[0] + x[1]).astype(np.uint32)
            x[1] = _rotl(x[1], r)
            x[1] = x[1] ^ x[0]
        x[0] = (x[0] + ks[(i + 1) % 3]).astype(np.uint32)
        x[1] = (x[1] + ks[(i + 2) % 3] + np.uint32(i + 1)).astype(np.uint32)
    return x[0], x[1]


def _np_fold_in(key, data):
    b1, b2 = _tf2x32(key[0], key[1], np.zeros(1, np.uint32),
                     np.array([data], np.uint32))
    return (b1[0], b2[0])


def _np_split2(key):
    b1, b2 = _tf2x32(key[0], key[1], np.zeros(2, np.uint32),
                     np.arange(2, dtype=np.uint32))
    return (b1[0], b2[0]), (b1[1], b2[1])


def _np_bits(key, n):
    b1, b2 = _tf2x32(key[0], key[1], np.zeros(n, np.uint32),
                     np.arange(n, dtype=np.uint32))
    return b1 ^ b2


def _build_tables():
    out = []
    for cls in (0, 1):
        k0 = _np_fold_in((np.uint32(0), np.uint32(42)), cls)
        k1, sub1 = _np_split2(k0)
        bits1 = _np_bits(sub1, NANCH)
        _, sub2 = _np_split2(k1)
        bits2 = _np_bits(sub2, NANCH)
        for b in (bits1, bits2):
            sigma = np.argsort(b, kind="stable").astype(np.int32)
            g = np.empty(NANCH, np.int32)
            g[sigma] = np.arange(NANCH, dtype=np.int32)
            sig_pad = np.full(PAD, 2**30, np.int32)
            sig_pad[:NANCH] = sigma
            g_pad = np.zeros(PAD, np.int32)
            g_pad[:NANCH] = g
            out.append(sig_pad)
            out.append(g_pad)
    return np.stack(out)


_TABLES = _build_tables()
S1P, G1P, S2P, G2P, S1N, G1N, S2N, G2N = range(8)


def _iou_j(bbox_ref, j, ax1, ay1, ax2, ay2, area1):
    bx1 = bbox_ref[j, 0]
    by1 = bbox_ref[j, 1]
    bx2 = bbox_ref[j, 2]
    by2 = bbox_ref[j, 3]
    ix = jnp.maximum(jnp.minimum(ax2, bx2) - jnp.maximum(ax1, bx1), 0.0)
    iy = jnp.maximum(jnp.minimum(ay2, by2) - jnp.maximum(ay1, by1), 0.0)
    ia = ix * iy
    union = area1 + (bx2 - bx1) * (by2 - by1) - ia
    return ia / union, (bx1, by1, bx2, by2)


def _tc_a_body(bbox_ref, a_ref, label_ref, win_ref):
    ax1 = a_ref[0]
    ay1 = a_ref[1]
    ax2 = a_ref[2]
    ay2 = a_ref[3]
    keep = (ax1 >= 0.0) & (ay1 >= 0.0) & (ax2 < 1.0) & (ay2 < 1.0)
    area1 = (ax2 - ax1) * (ay2 - ay1)
    ridx = (lax.broadcasted_iota(jnp.int32, (ROWS, 128), 0) * 128
            + lax.broadcasted_iota(jnp.int32, (ROWS, 128), 1))
    best = jnp.full((ROWS, 128), -jnp.inf, jnp.float32)
    M = jnp.full((NBOX, 128), -2.0, jnp.float32)
    I = jnp.full((NBOX, 128), PAD, jnp.int32)
    mrow = lax.broadcasted_iota(jnp.int32, (NBOX, 128), 0)
    for j in range(NBOX):
        iou, _ = _iou_j(bbox_ref, j, ax1, ay1, ax2, ay2, area1)
        best = jnp.maximum(best, iou)
        miou = jnp.where(keep, iou, -1.0)
        lanemax = jnp.max(miou, axis=0, keepdims=True)
        cand = jnp.where(miou == lanemax, ridx, PAD)
        laneidx = jnp.min(cand, axis=0, keepdims=True)
        M = jnp.where(mrow == j, lanemax, M)
        I = jnp.where(mrow == j, laneidx, I)
    bmax = jnp.max(M, axis=1, keepdims=True)
    wvec = jnp.min(jnp.where(M == bmax, I, PAD), axis=1, keepdims=True)
    label = jnp.where(keep & (best < 0.3), 0, -1)
    label = jnp.where(keep & (best > 0.7), 1, label)
    label_ref[...] = label
    wlane = jnp.reshape(wvec, (1, NBOX))
    win_ref[...] = jnp.concatenate(
        [jnp.concatenate([wlane, jnp.zeros((1, 96), jnp.int32)], axis=1),
         jnp.zeros((7, 128), jnp.int32)], axis=0)


def _tc_b_body(bbox_ref, a_ref, loc_ref):
    ax1 = a_ref[0]
    ay1 = a_ref[1]
    ax2 = a_ref[2]
    ay2 = a_ref[3]
    keep = (ax1 >= 0.0) & (ay1 >= 0.0) & (ax2 < 1.0) & (ay2 < 1.0)
    area1 = (ax2 - ax1) * (ay2 - ay1)
    best = jnp.full((ROWS, 128), -jnp.inf, jnp.float32)
    zero = jnp.zeros((ROWS, 128), jnp.float32)
    sx1, sy1, sx2, sy2 = zero, zero, zero, zero
    for j in range(NBOX):
        iou, (bx1, by1, bx2, by2) = _iou_j(bbox_ref, j, ax1, ay1, ax2, ay2,
                                           area1)
        better = iou > best
        best = jnp.where(better, iou, best)
        sx1 = jnp.where(better, bx1, sx1)
        sy1 = jnp.where(better, by1, sy1)
        sx2 = jnp.where(better, bx2, sx2)
        sy2 = jnp.where(better, by2, sy2)
    aw = ax2 - ax1
    ah = ay2 - ay1
    fkeep = keep.astype(jnp.float32)
    loc_ref[0] = jnp.where(keep, ((sx1 + sx2) / 2.0 - (ax1 + ax2) / 2.0) / aw, 0.0)
    loc_ref[1] = jnp.where(keep, ((sy1 + sy2) / 2.0 - (ay1 + ay2) / 2.0) / ah, 0.0)
    loc_ref[2] = fkeep * jnp.log(jnp.where(keep, (sx2 - sx1) / aw, 1.0))
    loc_ref[3] = fkeep * jnp.log(jnp.where(keep, (sy2 - sy1) / ah, 1.0))


_tc_a_call = pl.pallas_call(
    _tc_a_body,
    in_specs=[
        pl.BlockSpec(memory_space=pltpu.SMEM),
        pl.BlockSpec(memory_space=pltpu.VMEM),
    ],
    out_specs=[
        pl.BlockSpec(memory_space=pltpu.VMEM),
        pl.BlockSpec(memory_space=pltpu.VMEM),
    ],
    out_shape=[
        jax.ShapeDtypeStruct((ROWS, 128), jnp.int32),
        jax.ShapeDtypeStruct((8, 128), jnp.int32),
    ],
)

_tc_b_call = pl.pallas_call(
    _tc_b_body,
    in_specs=[
        pl.BlockSpec(memory_space=pltpu.SMEM),
        pl.BlockSpec(memory_space=pltpu.VMEM),
    ],
    out_specs=[pl.BlockSpec(memory_space=pltpu.VMEM)],
    out_shape=[jax.ShapeDtypeStruct((4, ROWS, 128), jnp.float32)],
)


def _splat(x):
    return jnp.zeros((16,), jnp.int32) + x


def _sc_body(lbl_hbm, win_hbm, tbl_hbm, out_hbm,
             lbl_v, win_v, prank_v, nrank_v, siga_v, sigb_v,
             c1p_v, c2p_v, c1n_v, c2n_v,
             bg1_v, bc1_v, bg2_v, bc2_v,
             row_v, tbl_v,
             spm_cnt, spm_tot, spm_c1, spm_c2, spm_c1p, spm_c2p,
             sem1, sem2):
    t = lax.axis_index("s")
    base = t * CHUNK
    iota = lax.iota(jnp.int32, 16)
    zeros = _splat(0)
    ones = _splat(1)

    cp1 = pltpu.async_copy(tbl_hbm.at[G1N], bg1_v, sem1)
    cp2 = pltpu.async_copy(tbl_hbm.at[G2N], bg2_v, sem2)

    pltpu.sync_copy(lbl_hbm.at[pl.ds(base, CHUNK)], lbl_v)
    pltpu.sync_copy(win_hbm.at[0], win_v)

    for h in range(2):
        w = win_v[pl.ds(h * 16, 16)]
        loc = w - _splat(base)
        msk = (loc >= zeros) & (loc < _splat(CHUNK))
        locc = jnp.minimum(jnp.maximum(loc, zeros), _splat(CHUNK - 1))
        plsc.store_scatter(lbl_v, [locc], ones, mask=msk)

    def cnt_body(i, carry):
        cp, cn = carry
        la = lbl_v[pl.ds(i * 32, 16)]
        lb = lbl_v[pl.ds(i * 32 + 16, 16)]
        mpa = la == ones
        mna = la == zeros
        mpb = lb == ones
        mnb = lb == zeros
        pa = plsc.all_reduce_population_count(mpa)
        na = plsc.all_reduce_population_count(mna)
        pb = plsc.all_reduce_population_count(mpb)
        nb = plsc.all_reduce_population_count(mnb)
        prank_v[pl.ds(i * 32, 16)] = plsc.cumsum(jnp.where(mpa, ones, zeros)) + cp
        nrank_v[pl.ds(i * 32, 16)] = plsc.cumsum(jnp.where(mna, ones, zeros)) + cn
        prank_v[pl.ds(i * 32 + 16, 16)] = (
            plsc.cumsum(jnp.where(mpb, ones, zeros)) + cp + pa)
        nrank_v[pl.ds(i * 32 + 16, 16)] = (
            plsc.cumsum(jnp.where(mnb, ones, zeros)) + cn + na)
        return (cp + pa + pb, cn + na + nb)

    cp, cn = lax.fori_loop(0, VITER // 2, cnt_body, (zeros, zeros))
    my_pos = jnp.max(cp)
    my_neg = jnp.max(cn)
    row_v[...] = (jnp.where(iota == zeros, _splat(my_pos), zeros)
                  + jnp.where(iota == ones, _splat(my_neg), zeros))
    pltpu.sync_copy(row_v, spm_cnt.at[t])
    plsc.subcore_barrier()

    pltpu.sync_copy(spm_cnt, tbl_v)
    pc = plsc.load_gather(tbl_v, [iota, zeros])
    nc = plsc.load_gather(tbl_v, [iota, ones])
    n_pos = jnp.sum(pc)
    n_neg = jnp.sum(nc)
    tv = _splat(t)
    off_pos = jnp.sum(jnp.where(iota < tv, pc, zeros))
    off_neg = jnp.sum(jnp.where(iota < tv, nc, zeros))
    pos_active = n_pos > 128

    def c_pair(sa, sb, ca_v, cb_v, n):
        pltpu.sync_copy(tbl_hbm.at[sa, pl.ds(base, CHUNK)], siga_v)
        pltpu.sync_copy(tbl_hbm.at[sb, pl.ds(base, CHUNK)], sigb_v)
        nv = _splat(n)

        def body(i, carry):
            ca, cb = carry
            sa0 = siga_v[pl.ds(i * 32, 16)]
            sa1 = siga_v[pl.ds(i * 32 + 16, 16)]
            sb0 = sigb_v[pl.ds(i * 32, 16)]
            sb1 = sigb_v[pl.ds(i * 32 + 16, 16)]
            ma0 = sa0 < nv
            ma1 = sa1 < nv
            mb0 = sb0 < nv
            mb1 = sb1 < nv
            pa0 = plsc.all_reduce_population_count(ma0)
            pa1 = plsc.all_reduce_population_count(ma1)
            pb0 = plsc.all_reduce_population_count(mb0)
            pb1 = plsc.all_reduce_population_count(mb1)
            ia0 = jnp.where(ma0, ones, zeros)
            ia1 = jnp.where(ma1, ones, zeros)
            ib0 = jnp.where(mb0, ones, zeros)
            ib1 = jnp.where(mb1, ones, zeros)
            ca_v[pl.ds(i * 32, 16)] = plsc.cumsum(ia0) + ca - ia0
            cb_v[pl.ds(i * 32, 16)] = plsc.cumsum(ib0) + cb - ib0
            ca_v[pl.ds(i * 32 + 16, 16)] = plsc.cumsum(ia1) + (ca + pa0) - ia1
            cb_v[pl.ds(i * 32 + 16, 16)] = plsc.cumsum(ib1) + (cb + pb0) - ib1
            return (ca + pa0 + pa1, cb + pb0 + pb1)

        ta, tb = lax.fori_loop(0, VITER // 2, body, (zeros, zeros))
        return jnp.max(ta), jnp.max(tb)

    t1n, t2n = c_pair(S1N, S2N, c1n_v, c2n_v, n_neg)
    row_v[...] = (jnp.where(iota == _splat(2), _splat(t1n), zeros)
                  + jnp.where(iota == _splat(3), _splat(t2n), zeros))

    @pl.when(pos_active)
    def _():
        t1p, t2p = c_pair(S1P, S2P, c1p_v, c2p_v, n_pos)
        row_v[...] = (row_v[...]
                      + jnp.where(iota == zeros, _splat(t1p), zeros)
                      + jnp.where(iota == ones, _splat(t2p), zeros))

    pltpu.sync_copy(row_v, spm_tot.at[t])
    plsc.subcore_barrier()

    pltpu.sync_copy(spm_tot, tbl_v)

    def off_of(col):
        tc = plsc.load_gather(tbl_v, [iota, _splat(col)])
        return _splat(jnp.sum(jnp.where(iota < tv, tc, zeros)))

    o1n = off_of(2)
    o2n = off_of(3)

    def pub_body(i, carry):
        c1n_v[pl.ds(i * 16, 16)] = c1n_v[pl.ds(i * 16, 16)] + o1n
        c2n_v[pl.ds(i * 16, 16)] = c2n_v[pl.ds(i * 16, 16)] + o2n
        return carry

    lax.fori_loop(0, VITER, pub_body, 0)
    pltpu.sync_copy(c1n_v, spm_c1.at[pl.ds(base, CHUNK)])
    pltpu.sync_copy(c2n_v, spm_c2.at[pl.ds(base, CHUNK)])

    @pl.when(pos_active)
    def _():
        o1p = off_of(0)
        o2p = off_of(1)

        def body(i, carry):
            c1p_v[pl.ds(i * 16, 16)] = c1p_v[pl.ds(i * 16, 16)] + o1p
            c2p_v[pl.ds(i * 16, 16)] = c2p_v[pl.ds(i * 16, 16)] + o2p
            return carry

        lax.fori_loop(0, VITER, body, 0)
        pltpu.sync_copy(c1p_v, spm_c1p.at[pl.ds(base, CHUNK)])
        pltpu.sync_copy(c2p_v, spm_c2p.at[pl.ds(base, CHUNK)])

    plsc.subcore_barrier()

    maxi = _splat(PAD - 1)
    cp1.wait()
    cp2.wait()

    def drop_loop(rank_v, n, off, lblval, start):
        nvec = _splat(n)
        startv = _splat(start)
        offv = _splat(off)
        lv = _splat(lblval)

        two_round = nvec > _splat(1625)
        neg1 = _splat(-1)

        def chain(r):
            rc = jnp.minimum(jnp.maximum(r, zeros), maxi)
            a = plsc.load_gather(bg1_v, [rc])
            p1 = plsc.load_gather(bc1_v, [a])
            b = plsc.load_gather(bg2_v, [jnp.minimum(p1, maxi)])
            v2 = plsc.load_gather(bc2_v, [b])
            return jnp.where(two_round, v2, p1)

        def body(i, carry):
            l0 = lbl_v[pl.ds(i * 32, 16)]
            l1 = lbl_v[pl.ds(i * 32 + 16, 16)]
            r0 = rank_v[pl.ds(i * 32, 16)] - ones + offv
            r1 = rank_v[pl.ds(i * 32 + 16, 16)] - ones + offv
            v0 = chain(r0)
            v1 = chain(r1)
            d0 = (l0 == lv) & (v0 >= startv)
            d1 = (l1 == lv) & (v1 >= startv)
            lbl_v[pl.ds(i * 32, 16)] = jnp.where(d0, neg1, l0)
            lbl_v[pl.ds(i * 32 + 16, 16)] = jnp.where(d1, neg1, l1)
            return carry

        lax.fori_loop(0, VITER // 2, body, 0)

    @pl.when(pos_active)
    def _():
        pltpu.sync_copy(tbl_hbm.at[G1P], bg1_v)
        pltpu.sync_copy(spm_c1p, bc1_v)
        pltpu.sync_copy(tbl_hbm.at[G2P], bg2_v)
        pltpu.sync_copy(spm_c2p, bc2_v)
        drop_loop(prank_v, n_pos, off_pos, 1, 128)
        pltpu.sync_copy(tbl_hbm.at[G1N], bg1_v)
        pltpu.sync_copy(tbl_hbm.at[G2N], bg2_v)

    pltpu.sync_copy(spm_c1, bc1_v)
    pltpu.sync_copy(spm_c2, bc2_v)
    s = 256 - n_pos - n_neg
    start_lt = jnp.where(s >= 0, jnp.minimum(s, n_neg),
                         jnp.maximum(n_neg + s, 0))
    start_neg = jnp.where(n_pos >= 128, 128, start_lt)

    @pl.when(n_neg > 128)
    def _():
        drop_loop(nrank_v, n_neg, off_neg, 0, start_neg)

    pltpu.sync_copy(lbl_v, out_hbm.at[pl.ds(base, CHUNK)])


_sc_call = pl.kernel(
    _sc_body,
    mesh=plsc.VectorSubcoreMesh(core_axis_name="c", subcore_axis_name="s",
                                num_cores=1),
    out_type=jax.ShapeDtypeStruct((PAD,), jnp.int32),
    compiler_params=pltpu.CompilerParams(needs_layout_passes=False),
    scratch_types=[
        pltpu.VMEM((CHUNK,), jnp.int32),
        pltpu.VMEM((128,), jnp.int32),
        pltpu.VMEM((CHUNK,), jnp.int32),
        pltpu.VMEM((CHUNK,), jnp.int32),
        pltpu.VMEM((CHUNK,), jnp.int32),
        pltpu.VMEM((CHUNK,), jnp.int32),
        pltpu.VMEM((CHUNK,), jnp.int32),
        pltpu.VMEM((CHUNK,), jnp.int32),
        pltpu.VMEM((CHUNK,), jnp.int32),
        pltpu.VMEM((CHUNK,), jnp.int32),
        pltpu.VMEM((PAD,), jnp.int32),
        pltpu.VMEM((PAD,), jnp.int32),
        pltpu.VMEM((PAD,), jnp.int32),
        pltpu.VMEM((PAD,), jnp.int32),
        pltpu.VMEM((16,), jnp.int32),
        pltpu.VMEM((16, 16), jnp.int32),
        pltpu.VMEM_SHARED((NTILE, 16), jnp.int32),
        pltpu.VMEM_SHARED((NTILE, 16), jnp.int32),
        pltpu.VMEM_SHARED((PAD,), jnp.int32),
        pltpu.VMEM_SHARED((PAD,), jnp.int32),
        pltpu.VMEM_SHARED((PAD,), jnp.int32),
        pltpu.VMEM_SHARED((PAD,), jnp.int32),
        pltpu.SemaphoreType.DMA,
        pltpu.SemaphoreType.DMA,
    ],
)


def kernel(bbox, anchor):
    bbox = bbox.astype(jnp.float32)
    anchor = anchor.astype(jnp.float32)
    pads = jnp.tile(jnp.array([[2.0], [2.0], [2.1], [2.1]], jnp.float32),
                    (1, PAD - NANCH))
    acoord = jnp.concatenate([anchor.T, pads], axis=1).reshape(4, ROWS, 128)
    label0, winners = _tc_a_call(bbox, acoord)
    cls_pad = _sc_call(label0.reshape(PAD), winners, jnp.asarray(_TABLES))
    (locp,) = _tc_b_call(bbox, acoord)
    rpn_tg_cls = cls_pad[:NANCH]
    rpn_tg_loc = locp.reshape(4, PAD).T[:NANCH]
    return (rpn_tg_cls, rpn_tg_loc)


# speedup vs baseline: 1.1481x; 1.0053x over previous
import numpy as np
import jax
import jax.numpy as jnp
from jax import lax
from jax.experimental import pallas as pl
from jax.experimental.pallas import tpu as pltpu
from jax.experimental.pallas import tpu_sc as plsc

NBOX = 32
NANCH = 20000
PAD = 20480
ROWS = PAD // 128
NTILE = 16
CHUNK = PAD // NTILE
VITER = CHUNK // 16


def _rotl(x, r):
    r = np.uint32(r)
    return ((x << r) | (x >> np.uint32(32 - r))).astype(np.uint32)


def _tf2x32(k1, k2, x1, x2):
    x = [np.asarray(x1, np.uint32).copy(), np.asarray(x2, np.uint32).copy()]
    ks = [np.uint32(k1), np.uint32(k2),
          np.uint32(np.uint32(k1) ^ np.uint32(k2) ^ np.uint32(0x1BD11BDA))]
    rotations = [[13, 15, 26, 6], [17, 29, 16, 24]]
    x[0] = (x[0] + ks[0]).astype(np.uint32)
    x[1] = (x[1] + ks[1]).astype(np.uint32)
    for i in range(5):
        for r in rotations[i % 2]:
            x[0] = (x[0] + x[1]).astype(np.uint32)
            x[1] = _rotl(x[1], r)
            x[1] = x[1] ^ x[0]
        x[0] = (x[0] + ks[(i + 1) % 3]).astype(np.uint32)
        x[1] = (x[1] + ks[(i + 2) % 3] + np.uint32(i + 1)).astype(np.uint32)
    return x[0], x[1]


def _np_fold_in(key, data):
    b1, b2 = _tf2x32(key[0], key[1], np.zeros(1, np.uint32),
                     np.array([data], np.uint32))
    return (b1[0], b2[0])


def _np_split2(key):
    b1, b2 = _tf2x32(key[0], key[1], np.zeros(2, np.uint32),
                     np.arange(2, dtype=np.uint32))
    return (b1[0], b2[0]), (b1[1], b2[1])


def _np_bits(key, n):
    b1, b2 = _tf2x32(key[0], key[1], np.zeros(n, np.uint32),
                     np.arange(n, dtype=np.uint32))
    return b1 ^ b2


def _build_tables():
    out = []
    for cls in (0, 1):
        k0 = _np_fold_in((np.uint32(0), np.uint32(42)), cls)
        k1, sub1 = _np_split2(k0)
        bits1 = _np_bits(sub1, NANCH)
        _, sub2 = _np_split2(k1)
        bits2 = _np_bits(sub2, NANCH)
        for b in (bits1, bits2):
            sigma = np.argsort(b, kind="stable").astype(np.int32)
            g = np.empty(NANCH, np.int32)
            g[sigma] = np.arange(NANCH, dtype=np.int32)
            sig_pad = np.full(PAD, 2**30, np.int32)
            sig_pad[:NANCH] = sigma
            g_pad = np.zeros(PAD, np.int32)
            g_pad[:NANCH] = g
            out.append(sig_pad)
            out.append(g_pad)
    return np.stack(out)


_TABLES = _build_tables()
S1P, G1P, S2P, G2P, S1N, G1N, S2N, G2N = range(8)


def _iou_j(bbox_ref, j, ax1, ay1, ax2, ay2, area1):
    bx1 = bbox_ref[j, 0]
    by1 = bbox_ref[j, 1]
    bx2 = bbox_ref[j, 2]
    by2 = bbox_ref[j, 3]
    ix = jnp.maximum(jnp.minimum(ax2, bx2) - jnp.maximum(ax1, bx1), 0.0)
    iy = jnp.maximum(jnp.minimum(ay2, by2) - jnp.maximum(ay1, by1), 0.0)
    ia = ix * iy
    union = area1 + (bx2 - bx1) * (by2 - by1) - ia
    return ia / union, (bx1, by1, bx2, by2)


def _tc_a_body(bbox_ref, a_ref, label_ref, win_ref):
    ax1 = a_ref[0]
    ay1 = a_ref[1]
    ax2 = a_ref[2]
    ay2 = a_ref[3]
    keep = (ax1 >= 0.0) & (ay1 >= 0.0) & (ax2 < 1.0) & (ay2 < 1.0)
    area1 = (ax2 - ax1) * (ay2 - ay1)
    ridx = (lax.broadcasted_iota(jnp.int32, (ROWS, 128), 0) * 128
            + lax.broadcasted_iota(jnp.int32, (ROWS, 128), 1))
    best = jnp.full((ROWS, 128), -jnp.inf, jnp.float32)
    M = jnp.full((NBOX, 128), -2.0, jnp.float32)
    I = jnp.full((NBOX, 128), PAD, jnp.int32)
    mrow = lax.broadcasted_iota(jnp.int32, (NBOX, 128), 0)
    for j in range(NBOX):
        iou, _ = _iou_j(bbox_ref, j, ax1, ay1, ax2, ay2, area1)
        best = jnp.maximum(best, iou)
        miou = jnp.where(keep, iou, -1.0)
        lanemax = jnp.max(miou, axis=0, keepdims=True)
        cand = jnp.where(miou == lanemax, ridx, PAD)
        laneidx = jnp.min(cand, axis=0, keepdims=True)
        M = jnp.where(mrow == j, lanemax, M)
        I = jnp.where(mrow == j, laneidx, I)
    bmax = jnp.max(M, axis=1, keepdims=True)
    wvec = jnp.min(jnp.where(M == bmax, I, PAD), axis=1, keepdims=True)
    label = jnp.where(keep & (best < 0.3), 0, -1)
    label = jnp.where(keep & (best > 0.7), 1, label)
    label_ref[...] = label
    wlane = jnp.reshape(wvec, (1, NBOX))
    win_ref[...] = jnp.concatenate(
        [jnp.concatenate([wlane, jnp.zeros((1, 96), jnp.int32)], axis=1),
         jnp.zeros((7, 128), jnp.int32)], axis=0)


def _tc_b_body(bbox_ref, a_ref, loc_ref):
    ax1 = a_ref[0]
    ay1 = a_ref[1]
    ax2 = a_ref[2]
    ay2 = a_ref[3]
    keep = (ax1 >= 0.0) & (ay1 >= 0.0) & (ax2 < 1.0) & (ay2 < 1.0)
    area1 = (ax2 - ax1) * (ay2 - ay1)
    best = jnp.full((ROWS, 128), -jnp.inf, jnp.float32)
    zero = jnp.zeros((ROWS, 128), jnp.float32)
    sx1, sy1, sx2, sy2 = zero, zero, zero, zero
    for j in range(NBOX):
        iou, (bx1, by1, bx2, by2) = _iou_j(bbox_ref, j, ax1, ay1, ax2, ay2,
                                           area1)
        better = iou > best
        best = jnp.where(better, iou, best)
        sx1 = jnp.where(better, bx1, sx1)
        sy1 = jnp.where(better, by1, sy1)
        sx2 = jnp.where(better, bx2, sx2)
        sy2 = jnp.where(better, by2, sy2)
    aw = ax2 - ax1
    ah = ay2 - ay1
    fkeep = keep.astype(jnp.float32)
    loc_ref[0] = jnp.where(keep, ((sx1 + sx2) / 2.0 - (ax1 + ax2) / 2.0) / aw, 0.0)
    loc_ref[1] = jnp.where(keep, ((sy1 + sy2) / 2.0 - (ay1 + ay2) / 2.0) / ah, 0.0)
    loc_ref[2] = fkeep * jnp.log(jnp.where(keep, (sx2 - sx1) / aw, 1.0))
    loc_ref[3] = fkeep * jnp.log(jnp.where(keep, (sy2 - sy1) / ah, 1.0))


_tc_a_call = pl.pallas_call(
    _tc_a_body,
    in_specs=[
        pl.BlockSpec(memory_space=pltpu.SMEM),
        pl.BlockSpec(memory_space=pltpu.VMEM),
    ],
    out_specs=[
        pl.BlockSpec(memory_space=pltpu.VMEM),
        pl.BlockSpec(memory_space=pltpu.VMEM),
    ],
    out_shape=[
        jax.ShapeDtypeStruct((ROWS, 128), jnp.int32),
        jax.ShapeDtypeStruct((8, 128), jnp.int32),
    ],
)

_tc_b_call = pl.pallas_call(
    _tc_b_body,
    in_specs=[
        pl.BlockSpec(memory_space=pltpu.SMEM),
        pl.BlockSpec(memory_space=pltpu.VMEM),
    ],
    out_specs=[pl.BlockSpec(memory_space=pltpu.VMEM)],
    out_shape=[jax.ShapeDtypeStruct((4, ROWS, 128), jnp.float32)],
)


def _splat(x):
    return jnp.zeros((16,), jnp.int32) + x


def _sc_body(lbl_hbm, win_hbm, tbl_hbm, out_hbm,
             lbl_v, win_v, prank_v, nrank_v, siga_v, sigb_v,
             c1p_v, c2p_v, c1n_v, c2n_v,
             bg1_v, bc1_v, bg2_v, bc2_v,
             row_v, off1_v, off2_v, tbl_v,
             spm_cnt, spm_tot, spm_c1, spm_c2, spm_c1p, spm_c2p,
             sem1, sem2):
    t = lax.axis_index("s")
    base = t * CHUNK
    iota = lax.iota(jnp.int32, 16)
    zeros = _splat(0)
    ones = _splat(1)

    cp1 = pltpu.async_copy(tbl_hbm.at[G1N], bg1_v, sem1)
    cp2 = pltpu.async_copy(tbl_hbm.at[G2N], bg2_v, sem2)

    pltpu.sync_copy(lbl_hbm.at[pl.ds(base, CHUNK)], lbl_v)
    pltpu.sync_copy(win_hbm.at[0], win_v)

    for h in range(2):
        w = win_v[pl.ds(h * 16, 16)]
        loc = w - _splat(base)
        msk = (loc >= zeros) & (loc < _splat(CHUNK))
        locc = jnp.minimum(jnp.maximum(loc, zeros), _splat(CHUNK - 1))
        plsc.store_scatter(lbl_v, [locc], ones, mask=msk)

    def cnt_body(i, carry):
        cp, cn = carry
        la = lbl_v[pl.ds(i * 32, 16)]
        lb = lbl_v[pl.ds(i * 32 + 16, 16)]
        mpa = la == ones
        mna = la == zeros
        mpb = lb == ones
        mnb = lb == zeros
        pa = plsc.all_reduce_population_count(mpa)
        na = plsc.all_reduce_population_count(mna)
        pb = plsc.all_reduce_population_count(mpb)
        nb = plsc.all_reduce_population_count(mnb)
        prank_v[pl.ds(i * 32, 16)] = plsc.cumsum(jnp.where(mpa, ones, zeros)) + cp
        nrank_v[pl.ds(i * 32, 16)] = plsc.cumsum(jnp.where(mna, ones, zeros)) + cn
        prank_v[pl.ds(i * 32 + 16, 16)] = (
            plsc.cumsum(jnp.where(mpb, ones, zeros)) + cp + pa)
        nrank_v[pl.ds(i * 32 + 16, 16)] = (
            plsc.cumsum(jnp.where(mnb, ones, zeros)) + cn + na)
        return (cp + pa + pb, cn + na + nb)

    cp, cn = lax.fori_loop(0, VITER // 2, cnt_body, (zeros, zeros))
    my_pos = jnp.max(cp)
    my_neg = jnp.max(cn)
    row_v[...] = (jnp.where(iota == zeros, _splat(my_pos), zeros)
                  + jnp.where(iota == ones, _splat(my_neg), zeros))
    pltpu.sync_copy(row_v, spm_cnt.at[t])
    plsc.subcore_barrier()

    pltpu.sync_copy(spm_cnt, tbl_v)
    pc = plsc.load_gather(tbl_v, [iota, zeros])
    nc = plsc.load_gather(tbl_v, [iota, ones])
    n_pos = jnp.sum(pc)
    n_neg = jnp.sum(nc)
    tv = _splat(t)
    off_pos = jnp.sum(jnp.where(iota < tv, pc, zeros))
    off_neg = jnp.sum(jnp.where(iota < tv, nc, zeros))
    pos_active = n_pos > 128

    def c_pair(sa, sb, ca_v, cb_v, n, spma, spmb):
        pltpu.sync_copy(tbl_hbm.at[sa, pl.ds(base, CHUNK)], siga_v)
        pltpu.sync_copy(tbl_hbm.at[sb, pl.ds(base, CHUNK)], sigb_v)
        nv = _splat(n)

        def body(i, carry):
            ca, cb = carry
            sa0 = siga_v[pl.ds(i * 32, 16)]
            sa1 = siga_v[pl.ds(i * 32 + 16, 16)]
            sb0 = sigb_v[pl.ds(i * 32, 16)]
            sb1 = sigb_v[pl.ds(i * 32 + 16, 16)]
            ma0 = sa0 < nv
            ma1 = sa1 < nv
            mb0 = sb0 < nv
            mb1 = sb1 < nv
            pa0 = plsc.all_reduce_population_count(ma0)
            pa1 = plsc.all_reduce_population_count(ma1)
            pb0 = plsc.all_reduce_population_count(mb0)
            pb1 = plsc.all_reduce_population_count(mb1)
            ia0 = jnp.where(ma0, ones, zeros)
            ia1 = jnp.where(ma1, ones, zeros)
            ib0 = jnp.where(mb0, ones, zeros)
            ib1 = jnp.where(mb1, ones, zeros)
            ca_v[pl.ds(i * 32, 16)] = plsc.cumsum(ia0) + ca - ia0
            cb_v[pl.ds(i * 32, 16)] = plsc.cumsum(ib0) + cb - ib0
            ca_v[pl.ds(i * 32 + 16, 16)] = plsc.cumsum(ia1) + (ca + pa0) - ia1
            cb_v[pl.ds(i * 32 + 16, 16)] = plsc.cumsum(ib1) + (cb + pb0) - ib1
            return (ca + pa0 + pa1, cb + pb0 + pb1)

        ta, tb = lax.fori_loop(0, VITER // 2, body, (zeros, zeros))
        pltpu.sync_copy(ca_v, spma.at[pl.ds(base, CHUNK)])
        pltpu.sync_copy(cb_v, spmb.at[pl.ds(base, CHUNK)])
        return jnp.max(ta), jnp.max(tb)

    t1n, t2n = c_pair(S1N, S2N, c1n_v, c2n_v, n_neg, spm_c1, spm_c2)
    row_v[...] = (jnp.where(iota == _splat(2), _splat(t1n), zeros)
                  + jnp.where(iota == _splat(3), _splat(t2n), zeros))

    @pl.when(pos_active)
    def _():
        t1p, t2p = c_pair(S1P, S2P, c1p_v, c2p_v, n_pos, spm_c1p, spm_c2p)
        row_v[...] = (row_v[...]
                      + jnp.where(iota == zeros, _splat(t1p), zeros)
                      + jnp.where(iota == ones, _splat(t2p), zeros))

    pltpu.sync_copy(row_v, spm_tot.at[t])
    plsc.subcore_barrier()

    pltpu.sync_copy(spm_tot, tbl_v)

    def off_tbl(col, dst_v):
        tc = plsc.load_gather(tbl_v, [iota, _splat(col)])
        dst_v[...] = plsc.cumsum(tc) - tc

    maxi = _splat(PAD - 1)
    dmul = _splat(52429)
    cp1.wait()
    cp2.wait()

    def drop_loop(rank_v, n, off, lblval, start):
        nvec = _splat(n)
        startv = _splat(start)
        offv = _splat(off)
        lv = _splat(lblval)

        two_round = nvec > _splat(1625)
        neg1 = _splat(-1)

        def chain(r):
            rc = jnp.minimum(jnp.maximum(r, zeros), maxi)
            a = plsc.load_gather(bg1_v, [rc])
            p1 = (plsc.load_gather(bc1_v, [a])
                  + plsc.load_gather(off1_v, [(a * dmul) >> 26]))
            b = plsc.load_gather(bg2_v, [jnp.minimum(p1, maxi)])
            v2 = (plsc.load_gather(bc2_v, [b])
                  + plsc.load_gather(off2_v, [(b * dmul) >> 26]))
            return jnp.where(two_round, v2, p1)

        def body(i, carry):
            l0 = lbl_v[pl.ds(i * 32, 16)]
            l1 = lbl_v[pl.ds(i * 32 + 16, 16)]
            r0 = rank_v[pl.ds(i * 32, 16)] - ones + offv
            r1 = rank_v[pl.ds(i * 32 + 16, 16)] - ones + offv
            v0 = chain(r0)
            v1 = chain(r1)
            d0 = (l0 == lv) & (v0 >= startv)
            d1 = (l1 == lv) & (v1 >= startv)
            lbl_v[pl.ds(i * 32, 16)] = jnp.where(d0, neg1, l0)
            lbl_v[pl.ds(i * 32 + 16, 16)] = jnp.where(d1, neg1, l1)
            return carry

        lax.fori_loop(0, VITER // 2, body, 0)

    @pl.when(pos_active)
    def _():
        pltpu.sync_copy(tbl_hbm.at[G1P], bg1_v)
        pltpu.sync_copy(spm_c1p, bc1_v)
        pltpu.sync_copy(tbl_hbm.at[G2P], bg2_v)
        pltpu.sync_copy(spm_c2p, bc2_v)
        off_tbl(0, off1_v)
        off_tbl(1, off2_v)
        drop_loop(prank_v, n_pos, off_pos, 1, 128)
        pltpu.sync_copy(tbl_hbm.at[G1N], bg1_v)
        pltpu.sync_copy(tbl_hbm.at[G2N], bg2_v)

    cpc1 = pltpu.async_copy(spm_c1, bc1_v, sem1)
    cpc2 = pltpu.async_copy(spm_c2, bc2_v, sem2)
    off_tbl(2, off1_v)
    off_tbl(3, off2_v)
    cpc1.wait()
    cpc2.wait()
    s = 256 - n_pos - n_neg
    start_lt = jnp.where(s >= 0, jnp.minimum(s, n_neg),
                         jnp.maximum(n_neg + s, 0))
    start_neg = jnp.where(n_pos >= 128, 128, start_lt)

    @pl.when(n_neg > 128)
    def _():
        drop_loop(nrank_v, n_neg, off_neg, 0, start_neg)

    pltpu.sync_copy(lbl_v, out_hbm.at[pl.ds(base, CHUNK)])


_sc_call = pl.kernel(
    _sc_body,
    mesh=plsc.VectorSubcoreMesh(core_axis_name="c", subcore_axis_name="s",
                                num_cores=1),
    out_type=jax.ShapeDtypeStruct((PAD,), jnp.int32),
    compiler_params=pltpu.CompilerParams(needs_layout_passes=False),
    scratch_types=[
        pltpu.VMEM((CHUNK,), jnp.int32),
        pltpu.VMEM((128,), jnp.int32),
        pltpu.VMEM((CHUNK,), jnp.int32),
        pltpu.VMEM((CHUNK,), jnp.int32),
        pltpu.VMEM((CHUNK,), jnp.int32),
        pltpu.VMEM((CHUNK,), jnp.int32),
        pltpu.VMEM((CHUNK,), jnp.int32),
        pltpu.VMEM((CHUNK,), jnp.int32),
        pltpu.VMEM((CHUNK,), jnp.int32),
        pltpu.VMEM((CHUNK,), jnp.int32),
        pltpu.VMEM((PAD,), jnp.int32),
        pltpu.VMEM((PAD,), jnp.int32),
        pltpu.VMEM((PAD,), jnp.int32),
        pltpu.VMEM((PAD,), jnp.int32),
        pltpu.VMEM((16,), jnp.int32),
        pltpu.VMEM((16,), jnp.int32),
        pltpu.VMEM((16,), jnp.int32),
        pltpu.VMEM((16, 16), jnp.int32),
        pltpu.VMEM_SHARED((NTILE, 16), jnp.int32),
        pltpu.VMEM_SHARED((NTILE, 16), jnp.int32),
        pltpu.VMEM_SHARED((PAD,), jnp.int32),
        pltpu.VMEM_SHARED((PAD,), jnp.int32),
        pltpu.VMEM_SHARED((PAD,), jnp.int32),
        pltpu.VMEM_SHARED((PAD,), jnp.int32),
        pltpu.SemaphoreType.DMA,
        pltpu.SemaphoreType.DMA,
    ],
)


def kernel(bbox, anchor):
    bbox = bbox.astype(jnp.float32)
    anchor = anchor.astype(jnp.float32)
    pads = jnp.tile(jnp.array([[2.0], [2.0], [2.1], [2.1]], jnp.float32),
                    (1, PAD - NANCH))
    acoord = jnp.concatenate([anchor.T, pads], axis=1).reshape(4, ROWS, 128)
    label0, winners = _tc_a_call(bbox, acoord)
    cls_pad = _sc_call(label0.reshape(PAD), winners, jnp.asarray(_TABLES))
    (locp,) = _tc_b_call(bbox, acoord)
    rpn_tg_cls = cls_pad[:NANCH]
    rpn_tg_loc = locp.reshape(4, PAD).T[:NANCH]
    return (rpn_tg_cls, rpn_tg_loc)
